# baseline (speedup 1.0000x reference)
import jax
import jax.numpy as jnp
import numpy as np
from jax import lax
from jax.experimental import pallas as pl
from jax.experimental.pallas import tpu as pltpu
from jax.experimental.pallas import tpu_sc as plsc

N = 5000
M = 40000
NC = 2
NW = 32
CHUNK = 160
NP = NW * CHUNK
NG = CHUNK // 16
MAXC = 7
PIECE = 288
NPIECE = 4
TSPAN = PIECE * NPIECE
BIG = np.float32(3e38)


def _smooth_l1(a, b):
    d = jnp.abs(a - b)
    return jnp.where(d < 1.0, 0.5 * d * d, d - 0.5)


def _body(counts_hbm, pred_hbm, tgt_hbm, out_hbm,
          counts_v, pred_v, buf0_v, buf1_v, cmp_v, acc_v,
          psem, tsem0, tsem1):
    wid = lax.axis_index("s") * NC + lax.axis_index("c")
    iota = lax.broadcasted_iota(jnp.int32, (16,), 0)
    zeros16 = jnp.zeros((16,), jnp.int32)

    row_ofs = jnp.minimum(wid * CHUNK, N - CHUNK)
    drow = wid * CHUNK - row_ofs
    pred_cp = pltpu.async_copy(
        pred_hbm.at[pl.ds(row_ofs, CHUNK)], pred_v, psem)

    for j in range(8):
        counts_v[pl.ds(N - 8 + j * 16, 16)] = zeros16
    pltpu.sync_copy(counts_hbm, counts_v.at[pl.ds(0, N)])

    cbase = iota * CHUNK
    sA = zeros16
    sB = zeros16
    for j in range(CHUNK):
        sA = sA + plsc.load_gather(counts_v, [cbase + j])
        sB = sB + plsc.load_gather(counts_v, [cbase + (16 * CHUNK + j)])
    exclA = jnp.cumsum(sA) - sA
    exclB = jnp.cumsum(sB) - sB + jnp.sum(sA)
    widv = jnp.full((16,), wid, jnp.int32)
    pick = jnp.where(widv < 16, exclA, exclB)
    base = jnp.sum(jnp.where(iota == wid % 16, pick, zeros16))

    base8 = (base // 8) * 8
    bufs = [buf0_v, buf1_v]
    sems = [tsem0, tsem1]
    cps = [None] * NPIECE
    for i in range(2):
        cps[i] = pltpu.async_copy(
            tgt_hbm.at[pl.ds(base8 + i * PIECE, PIECE)], bufs[i], sems[i])

    carry = base - base8
    cvs, svs = [], []
    for g in range(NG):
        cv = counts_v[pl.ds(wid * CHUNK + g * 16, 16)]
        incl = jnp.cumsum(cv)
        cvs.append(cv)
        svs.append(carry + (incl - cv))
        carry = carry + jnp.sum(cv)

    for i in range(NPIECE):
        cps[i].wait()
        b = bufs[i % 2]
        for j in range(PIECE // 16):
            rows = jnp.full((16,), j * 16, jnp.int32) + iota
            for d in range(5):
                cmp_v[d, pl.ds(i * PIECE + j * 16, 16)] = \
                    plsc.load_gather(b, [rows, jnp.full((16,), d, jnp.int32)])
        if i + 2 < NPIECE:
            cps[i + 2] = pltpu.async_copy(
                tgt_hbm.at[pl.ds(base8 + (i + 2) * PIECE, PIECE)],
                bufs[i % 2], sems[i % 2])

    pred_cp.wait()
    accf = jnp.zeros((16,), jnp.float32)
    for g in range(NG):
        cv, sv = cvs[g], svs[g]
        prow = jnp.minimum(drow + g * 16 + iota, CHUNK - 1)
        p = [plsc.load_gather(pred_v, [prow, jnp.full((16,), d, jnp.int32)])
             for d in range(5)]
        cand = []
        for k in range(MAXC):
            rk = jnp.minimum(sv + k, TSPAN - 1)
            c = [plsc.load_gather(cmp_v, [jnp.full((16,), d, jnp.int32), rk])
                 for d in range(5)]
            pos = (p[0] - c[0]) * (p[0] - c[0]) + (p[1] - c[1]) * (p[1] - c[1])
            ang = (p[2] - c[2]) * (p[2] - c[2]) + (p[3] - c[3]) * (p[3] - c[3])
            wid_d = (p[4] - c[4]) * (p[4] - c[4])
            cand.append((jnp.where(cv > k, pos + ang + wid_d, BIG), rk))

        def merge(a, b):
            upd = b[0] < a[0]
            return (jnp.where(upd, b[0], a[0]), jnp.where(upd, b[1], a[1]))

        while len(cand) > 1:
            cand = [merge(cand[i2], cand[i2 + 1]) if i2 + 1 < len(cand)
                    else cand[i2] for i2 in range(0, len(cand), 2)]
        bestr = cand[0][1]
        c = [plsc.load_gather(cmp_v, [jnp.full((16,), d, jnp.int32), bestr])
             for d in range(5)]
        loss = (_smooth_l1(p[0], c[0]) + _smooth_l1(p[1], c[1])) \
            + (jnp.abs(p[2] - c[2]) + jnp.abs(p[3] - c[3])) \
            + _smooth_l1(p[4], c[4])
        accf = accf + jnp.where(cv > 0, loss, np.float32(0.0))

    acc_v[...] = accf
    pltpu.sync_copy(acc_v, out_hbm.at[wid])


def _make():
    return pl.kernel(
        _body,
        out_type=jax.ShapeDtypeStruct((NW, 16), jnp.float32),
        mesh=plsc.VectorSubcoreMesh(
            core_axis_name="c", subcore_axis_name="s",
            num_cores=NC, num_subcores=NW // NC),
        scratch_types=[
            pltpu.VMEM((NP,), jnp.int32),
            pltpu.VMEM((CHUNK, 6), jnp.float32),
            pltpu.VMEM((PIECE, 6), jnp.float32),
            pltpu.VMEM((PIECE, 6), jnp.float32),
            pltpu.VMEM((5, TSPAN), jnp.float32),
            pltpu.VMEM((16,), jnp.float32),
            pltpu.SemaphoreType.DMA,
            pltpu.SemaphoreType.DMA,
            pltpu.SemaphoreType.DMA,
        ],
        compiler_params=pltpu.CompilerParams(
            needs_layout_passes=False,
            skip_device_barrier=True,
            disable_bounds_checks=True,
            disable_semaphore_checks=True,
        ),
    )


@jax.jit
def kernel(pred, all_targets, target_counts):
    partial = _make()(target_counts, pred, all_targets)
    return jnp.sum(partial) / np.float32(N)


# speedup vs baseline: 1.6948x; 1.6948x over previous
import jax
import jax.numpy as jnp
import numpy as np
from jax import lax
from jax.experimental import pallas as pl
from jax.experimental.pallas import tpu as pltpu
from jax.experimental.pallas import tpu_sc as plsc

N = 5000
M = 40000
NC = 2
NW = 32
CHUNK = 160
NP = NW * CHUNK
NPP = 5248
NG = CHUNK // 16
MAXC = 7
TSPAN = 1280
PSPAN = 384
BIG = np.float32(3e38)


def _smooth_l1(a, b):
    d = jnp.abs(a - b)
    return jnp.where(d < 1.0, 0.5 * d * d, d - 0.5)


def _body(counts_hbm, pred_hbm, tgt_hbm, out_hbm,
          counts_v, pred_v, tgt_v, acc_v, sem):
    wid = lax.axis_index("s") * NC + lax.axis_index("c")
    iota = lax.broadcasted_iota(jnp.int32, (16,), 0)
    zeros16 = jnp.zeros((16,), jnp.int32)

    for j in range(8):
        counts_v[pl.ds(N - 8 + j * 16, 16)] = zeros16
    pltpu.sync_copy(counts_hbm, counts_v.at[pl.ds(0, N)])

    pofs = ((wid * CHUNK) // 128) * 128
    drow = wid * CHUNK - pofs
    pred_cp = pltpu.async_copy(pred_hbm.at[:, pl.ds(pofs, PSPAN)], pred_v, sem)

    cbase = iota * CHUNK
    sA = zeros16
    sB = zeros16
    for j in range(CHUNK):
        sA = sA + plsc.load_gather(counts_v, [cbase + j])
        sB = sB + plsc.load_gather(counts_v, [cbase + (16 * CHUNK + j)])
    exclA = jnp.cumsum(sA) - sA
    exclB = jnp.cumsum(sB) - sB + jnp.sum(sA)
    widv = jnp.full((16,), wid, jnp.int32)
    pick = jnp.where(widv < 16, exclA, exclB)
    base = jnp.sum(jnp.where(iota == wid % 16, pick, zeros16))

    base128 = (base // 128) * 128
    pltpu.sync_copy(tgt_hbm.at[:, pl.ds(base128, TSPAN)], tgt_v)
    pred_cp.wait()

    carry = base - base128
    accf = jnp.zeros((16,), jnp.float32)
    for g in range(NG):
        cv = counts_v[pl.ds(wid * CHUNK + g * 16, 16)]
        incl = jnp.cumsum(cv)
        sv = carry + (incl - cv)
        carry = carry + jnp.sum(cv)
        pcol = drow + g * 16 + iota
        p = [plsc.load_gather(pred_v, [jnp.full((16,), d, jnp.int32), pcol])
             for d in range(5)]
        bestd = jnp.full((16,), BIG, jnp.float32)
        bestr = zeros16
        for k in range(MAXC):
            rk = jnp.minimum(sv + k, TSPAN - 1)
            c = [plsc.load_gather(tgt_v, [jnp.full((16,), d, jnp.int32), rk])
                 for d in range(5)]
            pos = (p[0] - c[0]) * (p[0] - c[0]) + (p[1] - c[1]) * (p[1] - c[1])
            ang = (p[2] - c[2]) * (p[2] - c[2]) + (p[3] - c[3]) * (p[3] - c[3])
            wid_d = (p[4] - c[4]) * (p[4] - c[4])
            dist = jnp.where(cv > k, pos + ang + wid_d, BIG)
            upd = dist < bestd
            bestd = jnp.where(upd, dist, bestd)
            bestr = jnp.where(upd, rk, bestr)
        c = [plsc.load_gather(tgt_v, [jnp.full((16,), d, jnp.int32), bestr])
             for d in range(5)]
        loss = (_smooth_l1(p[0], c[0]) + _smooth_l1(p[1], c[1])) \
            + (jnp.abs(p[2] - c[2]) + jnp.abs(p[3] - c[3])) \
            + _smooth_l1(p[4], c[4])
        accf = accf + jnp.where(cv > 0, loss, np.float32(0.0))

    acc_v[...] = accf
    pltpu.sync_copy(acc_v, out_hbm.at[wid])


def _make():
    return pl.kernel(
        _body,
        out_type=jax.ShapeDtypeStruct((NW, 16), jnp.float32),
        mesh=plsc.VectorSubcoreMesh(
            core_axis_name="c", subcore_axis_name="s",
            num_cores=NC, num_subcores=NW // NC),
        scratch_types=[
            pltpu.VMEM((NP,), jnp.int32),
            pltpu.VMEM((8, PSPAN), jnp.float32),
            pltpu.VMEM((8, TSPAN), jnp.float32),
            pltpu.VMEM((16,), jnp.float32),
            pltpu.SemaphoreType.DMA,
        ],
        compiler_params=pltpu.CompilerParams(
            needs_layout_passes=False,
            skip_device_barrier=True,
            disable_bounds_checks=True,
            disable_semaphore_checks=True,
        ),
    )


@jax.jit
def kernel(pred, all_targets, target_counts):
    tgt_t = jnp.zeros((8, M), jnp.float32).at[:6].set(all_targets.T)
    pred_t = jnp.zeros((8, NPP), jnp.float32).at[:6, :N].set(pred.T)
    partial = _make()(target_counts, pred_t, tgt_t)
    return jnp.sum(partial) / np.float32(N)


# speedup vs baseline: 1.7882x; 1.0551x over previous
import jax
import jax.numpy as jnp
import numpy as np
from jax import lax
from jax.experimental import pallas as pl
from jax.experimental.pallas import tpu as pltpu
from jax.experimental.pallas import tpu_sc as plsc

N = 5000
M = 40000
NC = 2
NW = 32
CHUNK = 160
NP = NW * CHUNK
NPP = 5248
NG = CHUNK // 16
MAXC = 7
TSPAN = 1280
PSPAN = 384
BIG = np.float32(3e38)


def _smooth_l1(a, b):
    d = jnp.abs(a - b)
    return jnp.where(d < 1.0, 0.5 * d * d, d - 0.5)


def _body(counts_hbm, pred_hbm, tgt_hbm, out_hbm,
          counts_v, pred_v, tgt_v, acc_v, sem):
    wid = lax.axis_index("s") * NC + lax.axis_index("c")
    iota = lax.broadcasted_iota(jnp.int32, (16,), 0)
    zeros16 = jnp.zeros((16,), jnp.int32)

    for j in range(8):
        counts_v[pl.ds(N - 8 + j * 16, 16)] = zeros16
    pltpu.sync_copy(counts_hbm, counts_v.at[pl.ds(0, N)])

    pofs = ((wid * CHUNK) // 128) * 128
    drow = wid * CHUNK - pofs
    pred_cp = pltpu.async_copy(pred_hbm.at[:, pl.ds(pofs, PSPAN)], pred_v, sem)

    cbase = iota * CHUNK
    sA = zeros16
    sB = zeros16
    for j in range(CHUNK):
        pos = j + iota
        if j >= CHUNK - 15:
            pos = jnp.where(pos >= CHUNK, pos - CHUNK, pos)
        sA = sA + plsc.load_gather(counts_v, [cbase + pos])
        sB = sB + plsc.load_gather(counts_v, [cbase + (16 * CHUNK) + pos])
    exclA = jnp.cumsum(sA) - sA
    exclB = jnp.cumsum(sB) - sB + jnp.sum(sA)
    widv = jnp.full((16,), wid, jnp.int32)
    pick = jnp.where(widv < 16, exclA, exclB)
    base = jnp.sum(jnp.where(iota == wid % 16, pick, zeros16))

    base128 = (base // 128) * 128
    pltpu.sync_copy(tgt_hbm.at[:, pl.ds(base128, TSPAN)], tgt_v)
    pred_cp.wait()

    carry = base - base128
    accf = jnp.zeros((16,), jnp.float32)
    for g in range(NG):
        cv = counts_v[pl.ds(wid * CHUNK + g * 16, 16)]
        incl = jnp.cumsum(cv)
        sv = carry + (incl - cv)
        carry = carry + jnp.sum(cv)
        pcol = drow + g * 16 + iota
        p = [plsc.load_gather(pred_v, [jnp.full((16,), d, jnp.int32), pcol])
             for d in range(5)]
        bestd = jnp.full((16,), BIG, jnp.float32)
        bestr = zeros16
        for k in range(MAXC):
            rk = jnp.minimum(sv + k, TSPAN - 1)
            c = [plsc.load_gather(tgt_v, [jnp.full((16,), d, jnp.int32), rk])
                 for d in range(5)]
            pos = (p[0] - c[0]) * (p[0] - c[0]) + (p[1] - c[1]) * (p[1] - c[1])
            ang = (p[2] - c[2]) * (p[2] - c[2]) + (p[3] - c[3]) * (p[3] - c[3])
            wid_d = (p[4] - c[4]) * (p[4] - c[4])
            dist = jnp.where(cv > k, pos + ang + wid_d, BIG)
            upd = dist < bestd
            bestd = jnp.where(upd, dist, bestd)
            bestr = jnp.where(upd, rk, bestr)
        c = [plsc.load_gather(tgt_v, [jnp.full((16,), d, jnp.int32), bestr])
             for d in range(5)]
        loss = (_smooth_l1(p[0], c[0]) + _smooth_l1(p[1], c[1])) \
            + (jnp.abs(p[2] - c[2]) + jnp.abs(p[3] - c[3])) \
            + _smooth_l1(p[4], c[4])
        accf = accf + jnp.where(cv > 0, loss, np.float32(0.0))

    acc_v[...] = accf
    pltpu.sync_copy(acc_v, out_hbm.at[wid])


def _make():
    return pl.kernel(
        _body,
        out_type=jax.ShapeDtypeStruct((NW, 16), jnp.float32),
        mesh=plsc.VectorSubcoreMesh(
            core_axis_name="c", subcore_axis_name="s",
            num_cores=NC, num_subcores=NW // NC),
        scratch_types=[
            pltpu.VMEM((NP,), jnp.int32),
            pltpu.VMEM((6, PSPAN), jnp.float32),
            pltpu.VMEM((6, TSPAN), jnp.float32),
            pltpu.VMEM((16,), jnp.float32),
            pltpu.SemaphoreType.DMA,
        ],
        compiler_params=pltpu.CompilerParams(
            needs_layout_passes=False,
            skip_device_barrier=True,
            disable_bounds_checks=True,
            disable_semaphore_checks=True,
        ),
    )


@jax.jit
def kernel(pred, all_targets, target_counts):
    tgt_t = all_targets.T
    pred_t = jnp.zeros((6, NPP), jnp.float32).at[:, :N].set(pred.T)
    partial = _make()(target_counts, pred_t, tgt_t)
    return jnp.sum(partial) / np.float32(N)


# speedup vs baseline: 1.8794x; 1.0510x over previous
import jax
import jax.numpy as jnp
import numpy as np
from jax import lax
from jax.experimental import pallas as pl
from jax.experimental.pallas import tpu as pltpu
from jax.experimental.pallas import tpu_sc as plsc

N = 5000
M = 40000
NC = 2
NW = 32
CHUNK = 160
NP = NW * CHUNK
NPP = 5248
NG = CHUNK // 16
MAXC = 7
TSPAN = 1280
PSPAN = 384
BIG = np.float32(3e38)


def _smooth_l1(a, b):
    d = jnp.abs(a - b)
    return jnp.where(d < 1.0, 0.5 * d * d, d - 0.5)


def _body(counts_hbm, pred_hbm, tgt_hbm, out_hbm,
          counts_v, pred_v, tgt_v, acc_v, sem):
    wid = lax.axis_index("s") * NC + lax.axis_index("c")
    iota = lax.broadcasted_iota(jnp.int32, (16,), 0)
    zeros16 = jnp.zeros((16,), jnp.int32)

    for j in range(8):
        counts_v[pl.ds(N - 8 + j * 16, 16)] = zeros16
    pltpu.sync_copy(counts_hbm, counts_v.at[pl.ds(0, N)])

    pofs = ((wid * CHUNK) // 128) * 128
    drow = wid * CHUNK - pofs
    pred_cp = pltpu.async_copy(pred_hbm.at[:, pl.ds(pofs, PSPAN)], pred_v, sem)

    cbaseA = iota * CHUNK
    cbaseB = cbaseA + 16 * CHUNK

    def scan_step(t, s):
        sA, sB = s
        for jj in range(16):
            pos = t * 16 + jj + iota
            pos = jnp.where(pos >= CHUNK, pos - CHUNK, pos)
            sA = sA + plsc.load_gather(counts_v, [cbaseA + pos])
            sB = sB + plsc.load_gather(counts_v, [cbaseB + pos])
        return sA, sB

    sA, sB = lax.fori_loop(0, CHUNK // 16, scan_step, (zeros16, zeros16))
    exclA = jnp.cumsum(sA) - sA
    exclB = jnp.cumsum(sB) - sB + jnp.sum(sA)
    widv = jnp.full((16,), wid, jnp.int32)
    pick = jnp.where(widv < 16, exclA, exclB)
    base = jnp.sum(jnp.where(iota == wid % 16, pick, zeros16))

    base128 = (base // 128) * 128
    pltpu.sync_copy(tgt_hbm.at[:, pl.ds(base128, TSPAN)], tgt_v)
    pred_cp.wait()

    carry = base - base128
    accf = jnp.zeros((16,), jnp.float32)
    for g in range(NG):
        cv = counts_v[pl.ds(wid * CHUNK + g * 16, 16)]
        incl = jnp.cumsum(cv)
        sv = carry + (incl - cv)
        carry = carry + jnp.sum(cv)
        pcol = drow + g * 16 + iota
        p = [plsc.load_gather(pred_v, [jnp.full((16,), d, jnp.int32), pcol])
             for d in range(5)]
        bestd = jnp.full((16,), BIG, jnp.float32)
        bestr = zeros16
        for k in range(MAXC):
            rk = jnp.minimum(sv + k, TSPAN - 1)
            c = [plsc.load_gather(tgt_v, [jnp.full((16,), d, jnp.int32), rk])
                 for d in range(5)]
            pos = (p[0] - c[0]) * (p[0] - c[0]) + (p[1] - c[1]) * (p[1] - c[1])
            ang = (p[2] - c[2]) * (p[2] - c[2]) + (p[3] - c[3]) * (p[3] - c[3])
            wid_d = (p[4] - c[4]) * (p[4] - c[4])
            dist = jnp.where(cv > k, pos + ang + wid_d, BIG)
            upd = dist < bestd
            bestd = jnp.where(upd, dist, bestd)
            bestr = jnp.where(upd, rk, bestr)
        c = [plsc.load_gather(tgt_v, [jnp.full((16,), d, jnp.int32), bestr])
             for d in range(5)]
        loss = (_smooth_l1(p[0], c[0]) + _smooth_l1(p[1], c[1])) \
            + (jnp.abs(p[2] - c[2]) + jnp.abs(p[3] - c[3])) \
            + _smooth_l1(p[4], c[4])
        accf = accf + jnp.where(cv > 0, loss, np.float32(0.0))

    acc_v[...] = accf
    pltpu.sync_copy(acc_v, out_hbm.at[wid])


def _make():
    return pl.kernel(
        _body,
        out_type=jax.ShapeDtypeStruct((NW, 16), jnp.float32),
        mesh=plsc.VectorSubcoreMesh(
            core_axis_name="c", subcore_axis_name="s",
            num_cores=NC, num_subcores=NW // NC),
        scratch_types=[
            pltpu.VMEM((NP,), jnp.int32),
            pltpu.VMEM((6, PSPAN), jnp.float32),
            pltpu.VMEM((6, TSPAN), jnp.float32),
            pltpu.VMEM((16,), jnp.float32),
            pltpu.SemaphoreType.DMA,
        ],
        compiler_params=pltpu.CompilerParams(
            needs_layout_passes=False,
            skip_device_barrier=True,
            disable_bounds_checks=True,
            disable_semaphore_checks=True,
        ),
    )


@jax.jit
def kernel(pred, all_targets, target_counts):
    tgt_t = all_targets.T
    pred_t = jnp.zeros((6, NPP), jnp.float32).at[:, :N].set(pred.T)
    partial = _make()(target_counts, pred_t, tgt_t)
    return jnp.sum(partial) / np.float32(N)


# speedup vs baseline: 1.9827x; 1.0549x over previous
import jax
import jax.numpy as jnp
import numpy as np
from jax import lax
from jax.experimental import pallas as pl
from jax.experimental.pallas import tpu as pltpu
from jax.experimental.pallas import tpu_sc as plsc

N = 5000
M = 40000
NC = 2
NW = 32
CHUNK = 160
NP = NW * CHUNK
NPP = 5248
NG = CHUNK // 16
MAXC = 7
TSPAN = 1280
PSPAN = 384
BIG = np.float32(3e38)


def _smooth_l1(a, b):
    d = jnp.abs(a - b)
    return jnp.where(d < 1.0, 0.5 * d * d, d - 0.5)


def _body(counts_hbm, pred_hbm, tgt_hbm, out_hbm,
          counts_v, pred_v, tgt_v, acc_v, sem):
    wid = lax.axis_index("s") * NC + lax.axis_index("c")
    iota = lax.broadcasted_iota(jnp.int32, (16,), 0)
    zeros16 = jnp.zeros((16,), jnp.int32)

    for j in range(8):
        counts_v[pl.ds(N - 8 + j * 16, 16)] = zeros16
    pltpu.sync_copy(counts_hbm, counts_v.at[pl.ds(0, N)])

    pofs = ((wid * CHUNK) // 128) * 128
    drow = wid * CHUNK - pofs
    pred_cp = pltpu.async_copy(pred_hbm.at[:, pl.ds(pofs, PSPAN)], pred_v, sem)

    cbaseA = iota * CHUNK
    cbaseB = cbaseA + 16 * CHUNK

    def scan_step(t, s):
        sA, sB = s
        for jj in range(16):
            pos = t * 16 + jj + iota
            pos = jnp.where(pos >= CHUNK, pos - CHUNK, pos)
            sA = sA + plsc.load_gather(counts_v, [cbaseA + pos])
            sB = sB + plsc.load_gather(counts_v, [cbaseB + pos])
        return sA, sB

    sA, sB = lax.fori_loop(0, CHUNK // 16, scan_step, (zeros16, zeros16))
    exclA = jnp.cumsum(sA) - sA
    exclB = jnp.cumsum(sB) - sB + jnp.sum(sA)
    widv = jnp.full((16,), wid, jnp.int32)
    pick = jnp.where(widv < 16, exclA, exclB)
    base = jnp.sum(jnp.where(iota == wid % 16, pick, zeros16))

    base128 = (base // 128) * 128
    pltpu.sync_copy(tgt_hbm.at[:, pl.ds(base128, TSPAN)], tgt_v)
    pred_cp.wait()

    def group_step(g, s):
        accf, carry = s
        cv = counts_v[pl.ds(wid * CHUNK + g * 16, 16)]
        incl = jnp.cumsum(cv)
        sv = carry + (incl - cv)
        pcol = drow + g * 16 + iota
        p = [plsc.load_gather(pred_v, [jnp.full((16,), d, jnp.int32), pcol])
             for d in range(5)]
        bestd = jnp.full((16,), BIG, jnp.float32)
        bestr = zeros16
        for k in range(MAXC):
            rk = jnp.minimum(sv + k, TSPAN - 1)
            c = [plsc.load_gather(tgt_v, [jnp.full((16,), d, jnp.int32), rk])
                 for d in range(5)]
            pos = (p[0] - c[0]) * (p[0] - c[0]) + (p[1] - c[1]) * (p[1] - c[1])
            ang = (p[2] - c[2]) * (p[2] - c[2]) + (p[3] - c[3]) * (p[3] - c[3])
            wid_d = (p[4] - c[4]) * (p[4] - c[4])
            dist = jnp.where(cv > k, pos + ang + wid_d, BIG)
            upd = dist < bestd
            bestd = jnp.where(upd, dist, bestd)
            bestr = jnp.where(upd, rk, bestr)
        c = [plsc.load_gather(tgt_v, [jnp.full((16,), d, jnp.int32), bestr])
             for d in range(5)]
        loss = (_smooth_l1(p[0], c[0]) + _smooth_l1(p[1], c[1])) \
            + (jnp.abs(p[2] - c[2]) + jnp.abs(p[3] - c[3])) \
            + _smooth_l1(p[4], c[4])
        accf = accf + jnp.where(cv > 0, loss, np.float32(0.0))
        return accf, carry + jnp.sum(cv)

    carry0 = base - base128
    accf, _ = lax.fori_loop(0, NG, group_step,
                            (jnp.zeros((16,), jnp.float32), carry0))

    acc_v[...] = accf
    pltpu.sync_copy(acc_v, out_hbm.at[wid])


def _make():
    return pl.kernel(
        _body,
        out_type=jax.ShapeDtypeStruct((NW, 16), jnp.float32),
        mesh=plsc.VectorSubcoreMesh(
            core_axis_name="c", subcore_axis_name="s",
            num_cores=NC, num_subcores=NW // NC),
        scratch_types=[
            pltpu.VMEM((NP,), jnp.int32),
            pltpu.VMEM((6, PSPAN), jnp.float32),
            pltpu.VMEM((6, TSPAN), jnp.float32),
            pltpu.VMEM((16,), jnp.float32),
            pltpu.SemaphoreType.DMA,
        ],
        compiler_params=pltpu.CompilerParams(
            needs_layout_passes=False,
            skip_device_barrier=True,
            disable_bounds_checks=True,
            disable_semaphore_checks=True,
        ),
    )


@jax.jit
def kernel(pred, all_targets, target_counts):
    tgt_t = all_targets.T
    pred_t = jnp.zeros((6, NPP), jnp.float32).at[:, :N].set(pred.T)
    partial = _make()(target_counts, pred_t, tgt_t)
    return jnp.sum(partial) / np.float32(N)
